# baseline (speedup 1.0000x reference)
import functools

import jax
import jax.numpy as jnp
from jax import lax
from jax.experimental import pallas as pl
from jax.experimental.pallas import tpu as pltpu
from jax.experimental.pallas import tpu_sc as plsc

BATCH = 16384
FIELDS = 26
DIM = 64
NUM_CORES = 2
NUM_SUBCORES = 16
NW = NUM_CORES * NUM_SUBCORES
TOTAL = BATCH * FIELDS
CHUNK = 128
CHUNKS = TOTAL // (NW * CHUNK)
ROWS_PER_W = CHUNKS * CHUNK
NBUF = 8
LOOKAHEAD = 4


def _make_kernel():
    mesh = plsc.VectorSubcoreMesh(core_axis_name="c", subcore_axis_name="s")

    @functools.partial(
        pl.kernel,
        mesh=mesh,
        out_type=jax.ShapeDtypeStruct((TOTAL, DIM), jnp.float32),
        scratch_types=(
            [pltpu.VMEM((ROWS_PER_W,), jnp.int32)]
            + [pltpu.VMEM((CHUNK, DIM), jnp.float32) for _ in range(NBUF)]
            + [pltpu.SemaphoreType.DMA((NBUF,)), pltpu.SemaphoreType.DMA((NBUF,))]
        ),
        compiler_params=pltpu.CompilerParams(use_tc_tiling_on_sc=False),
    )
    def body(x_hbm, w_hbm, out_hbm, idx_v, *rest):
        rows = rest[:NBUF]
        gsem, wsem = rest[NBUF], rest[NBUF + 1]
        wid = lax.axis_index("s") * NUM_CORES + lax.axis_index("c")
        base = wid * ROWS_PER_W
        pltpu.sync_copy(x_hbm.at[pl.ds(base, ROWS_PER_W)], idx_v)

        def fire_gather(c, b):
            pltpu.async_copy(
                w_hbm.at[idx_v.at[pl.ds(c * CHUNK, CHUNK)]], rows[b],
                gsem.at[b])

        def wait_gather(b):
            pltpu.make_async_copy(
                w_hbm.at[idx_v.at[pl.ds(0, CHUNK)]], rows[b],
                gsem.at[b]).wait()

        def fire_write(c, b):
            pltpu.async_copy(
                rows[b], out_hbm.at[pl.ds(base + c * CHUNK, CHUNK)],
                wsem.at[b])

        def wait_write(b):
            pltpu.make_async_copy(
                rows[b], out_hbm.at[pl.ds(base, CHUNK)], wsem.at[b]).wait()

        for b in range(LOOKAHEAD):
            fire_gather(b, b)

        for b in range(NBUF):
            wait_gather(b)
            fire_write(b, b)
            bb = (b + LOOKAHEAD) % NBUF
            if b >= NBUF - LOOKAHEAD:
                wait_write(bb)
            fire_gather(b + LOOKAHEAD, bb)

        def block(gi, carry):
            g = gi * NBUF
            for b in range(NBUF):
                c = g + b
                wait_gather(b)
                fire_write(c, b)
                bb = (b + LOOKAHEAD) % NBUF
                wait_write(bb)
                fire_gather(c + LOOKAHEAD, bb)
            return carry

        lax.fori_loop(1, CHUNKS // NBUF - 1, block, 0)

        g = CHUNKS - NBUF
        for b in range(NBUF):
            c = g + b
            wait_gather(b)
            fire_write(c, b)
            if b < LOOKAHEAD:
                bb = (b + LOOKAHEAD) % NBUF
                wait_write(bb)
                fire_gather(c + LOOKAHEAD, bb)

        for b in range(NBUF):
            wait_write(b)

    return body


_kern = _make_kernel()


def kernel(x, weight):
    xf = x.reshape(TOTAL).astype(jnp.int32)
    out = _kern(xf, weight)
    return out.reshape(BATCH, FIELDS, DIM)


# speedup vs baseline: 1.0044x; 1.0044x over previous
import functools

import jax
import jax.numpy as jnp
from jax import lax
from jax.experimental import pallas as pl
from jax.experimental.pallas import tpu as pltpu
from jax.experimental.pallas import tpu_sc as plsc

BATCH = 16384
FIELDS = 26
DIM = 64
NUM_CORES = 2
NUM_SUBCORES = 16
NW = NUM_CORES * NUM_SUBCORES
B_PER_W = BATCH // NW
CHUNK = 128
KBLK = B_PER_W // CHUNK
CHUNKS = FIELDS * KBLK
NBUF = 8
LOOKAHEAD = 4


def _make_kernel():
    mesh = plsc.VectorSubcoreMesh(core_axis_name="c", subcore_axis_name="s")

    @functools.partial(
        pl.kernel,
        mesh=mesh,
        out_type=jax.ShapeDtypeStruct((BATCH, FIELDS, DIM), jnp.float32),
        scratch_types=(
            [pltpu.VMEM((FIELDS, B_PER_W), jnp.int32)]
            + [pltpu.VMEM((CHUNK, DIM), jnp.float32) for _ in range(NBUF)]
            + [pltpu.SemaphoreType.DMA((NBUF,)), pltpu.SemaphoreType.DMA((NBUF,))]
        ),
        compiler_params=pltpu.CompilerParams(use_tc_tiling_on_sc=False),
    )
    def body(xt_hbm, w_hbm, out_hbm, idx_v, *rest):
        rows = rest[:NBUF]
        gsem, wsem = rest[NBUF], rest[NBUF + 1]
        wid = lax.axis_index("s") * NUM_CORES + lax.axis_index("c")
        base_b = wid * B_PER_W
        pltpu.sync_copy(xt_hbm.at[:, pl.ds(base_b, B_PER_W)], idx_v)

        def fire_gather(c, b):
            f = c // KBLK
            k = c % KBLK
            pltpu.async_copy(
                w_hbm.at[idx_v.at[f, pl.ds(k * CHUNK, CHUNK)]], rows[b],
                gsem.at[b])

        def wait_gather(b):
            pltpu.make_async_copy(
                w_hbm.at[idx_v.at[0, pl.ds(0, CHUNK)]], rows[b],
                gsem.at[b]).wait()

        def fire_write(c, b):
            f = c // KBLK
            k = c % KBLK
            pltpu.async_copy(
                rows[b], out_hbm.at[pl.ds(base_b + k * CHUNK, CHUNK), f],
                wsem.at[b])

        def wait_write(b):
            pltpu.make_async_copy(
                rows[b], out_hbm.at[pl.ds(0, CHUNK), 0], wsem.at[b]).wait()

        for b in range(LOOKAHEAD):
            fire_gather(b, b)

        for b in range(NBUF):
            wait_gather(b)
            fire_write(b, b)
            bb = (b + LOOKAHEAD) % NBUF
            if b >= NBUF - LOOKAHEAD:
                wait_write(bb)
            fire_gather(b + LOOKAHEAD, bb)

        def block(gi, carry):
            g = gi * NBUF
            for b in range(NBUF):
                c = g + b
                wait_gather(b)
                fire_write(c, b)
                bb = (b + LOOKAHEAD) % NBUF
                wait_write(bb)
                fire_gather(c + LOOKAHEAD, bb)
            return carry

        lax.fori_loop(1, CHUNKS // NBUF - 1, block, 0)

        g = CHUNKS - NBUF
        for b in range(NBUF):
            c = g + b
            wait_gather(b)
            fire_write(c, b)
            if b < LOOKAHEAD:
                bb = (b + LOOKAHEAD) % NBUF
                wait_write(bb)
                fire_gather(c + LOOKAHEAD, bb)

        for b in range(NBUF):
            wait_write(b)

    return body


_kern = _make_kernel()


def kernel(x, weight):
    xt = x.T.astype(jnp.int32)
    return _kern(xt, weight)
